# Initial kernel scaffold; baseline (speedup 1.0000x reference)
import jax
import jax.numpy as jnp
from jax.experimental import pallas as pl


def kernel(x, edge_index, edge_attr, W1, b1, W2, b2, W_root, b_nn, W_gat, att_src, att_dst, b_gat, W_fc1, b_fc1, W_fc2, b_fc2):
    raise NotImplementedError("write your pallas kernel here")


# speedup vs baseline: 5.8395x; 5.8395x over previous
import functools

import jax
import jax.numpy as jnp
from jax import lax
from jax.experimental import pallas as pl
from jax.experimental.pallas import tpu as pltpu
from jax.experimental.pallas import tpu_sc as plsc

N = 10000
E = 160000
NFI = 128
NFO = 32
MH = 32
GD = 16
HID = 256
FIN = 128

NC, NS = 2, 16
NW = NC * NS
NT = NS * 640
EP = NW * 5120
EPW = EP // NW
EPC = EP // NS
HALF = NT // 2
TD = 6144

f32 = jnp.float32
i32 = jnp.int32

_mesh = plsc.VectorSubcoreMesh(
    core_axis_name="c", subcore_axis_name="s", num_cores=NC, num_subcores=NS)


def _worker_id():
    return lax.axis_index("s") * NC + lax.axis_index("c")


def _zero_fill(ref, nrows, ncols):
    z = jnp.zeros((16,), f32)

    def row(r, _):
        for h in range(ncols // 16):
            ref[r, pl.ds(h * 16, 16)] = z
        return 0

    lax.fori_loop(0, nrows, row, 0)


@functools.partial(
    pl.kernel,
    out_type=jax.ShapeDtypeStruct((EP, NFI), f32),
    mesh=_mesh,
    scratch_types=[
        pltpu.VMEM((EPW,), i32),
        pltpu.VMEM((64, NFI), f32),
        pltpu.SemaphoreType.DMA,
    ],
    compiler_params=pltpu.CompilerParams(needs_layout_passes=False),
)
def _sc_gather_rows(x_hbm, src_hbm, out_hbm, idx_v, rows_v, sem):
    wid = _worker_id()
    base = wid * EPW
    pltpu.sync_copy(src_hbm.at[pl.ds(base, EPW)], idx_v)

    def body(j, _):
        off = j * 64
        pltpu.async_copy(
            x_hbm.at[idx_v.at[pl.ds(off, 64)]], rows_v, sem).wait()
        pltpu.sync_copy(rows_v, out_hbm.at[pl.ds(base + off, 64)])
        return 0

    lax.fori_loop(0, EPW // 64, body, 0)


@functools.partial(
    pl.kernel,
    out_type=jax.ShapeDtypeStruct((NC, TD, 128), f32),
    mesh=_mesh,
    scratch_types=[
        pltpu.VMEM((1, 128), i32),
        pltpu.VMEM((128, 128), f32),
        pltpu.VMEM((64, 128), f32),
        pltpu.VMEM_SHARED((TD, 128), f32),
        pltpu.SemaphoreType.DMA,
    ],
    compiler_params=pltpu.CompilerParams(needs_layout_passes=False),
)
def _sc_scatter_rows(rows_hbm, idx2_hbm, out_hbm, idxc_v, rows_v, zb_v, table,
                     sem):
    cid = lax.axis_index("c")
    sid = lax.axis_index("s")
    stripe = TD // NS
    _zero_fill(zb_v, 64, 128)

    def zrow(k, _):
        pltpu.sync_copy(zb_v, table.at[pl.ds(sid * stripe + k * 64, 64)])
        return 0

    lax.fori_loop(0, stripe // 64, zrow, 0)
    plsc.subcore_barrier()
    base = sid * EPC

    def body(j, _):
        off = base + j * 128
        pltpu.sync_copy(idx2_hbm.at[cid, pl.ds(off, 128)], idxc_v.at[0])
        pltpu.sync_copy(rows_hbm.at[pl.ds(off, 128)], rows_v)
        pltpu.sync_copy(rows_v, table.at[idxc_v.at[0]], add=True)
        return 0

    lax.fori_loop(0, EPC // 128, body, 0)
    plsc.subcore_barrier()

    def wrow(k, _):
        off = sid * stripe + k * 64
        pltpu.sync_copy(table.at[pl.ds(off, 64)], rows_v.at[pl.ds(0, 64)])
        pltpu.sync_copy(rows_v.at[pl.ds(0, 64)],
                        out_hbm.at[cid, pl.ds(off, 64)])
        return 0

    lax.fori_loop(0, stripe // 64, wrow, 0)


@functools.partial(
    pl.kernel,
    out_type=jax.ShapeDtypeStruct((EP, 128), f32),
    mesh=_mesh,
    scratch_types=[
        pltpu.VMEM((NT,), f32),
        pltpu.VMEM((16,), f32),
        pltpu.VMEM((1, 128), i32),
        pltpu.VMEM((1, 128), i32),
        pltpu.VMEM((128, 128), f32),
        pltpu.SemaphoreType.DMA,
    ],
    compiler_params=pltpu.CompilerParams(needs_layout_passes=False),
)
def _sc_gat_rows(xwP_hbm, adP_hbm, c_hbm, si_hbm, di_hbm, out_hbm,
                 ad_v, c_v, si_v, di_v, xwr_v, sem):
    wid = _worker_id()
    base = wid * EPW
    pltpu.sync_copy(adP_hbm, ad_v)
    pltpu.sync_copy(c_hbm, c_v)
    cvec = c_v[...]
    iota16 = lax.iota(i32, 16)

    def body(j, _):
        off = base + j * 128
        pltpu.sync_copy(si_hbm.at[pl.ds(off, 128)], si_v.at[0])
        pltpu.sync_copy(di_hbm.at[pl.ds(off, 128)], di_v.at[0])
        pltpu.async_copy(xwP_hbm.at[si_v.at[0]], xwr_v, sem).wait()
        for g in range(8):
            e16 = iota16 + g * 16
            d16 = di_v[0, pl.ds(g * 16, 16)]
            asg = plsc.load_gather(xwr_v, [e16, jnp.full((16,), GD, i32)])
            adg = plsc.load_gather(ad_v, [d16])
            u = asg + adg
            pe = jnp.exp(jnp.maximum(u, 0.2 * u) - cvec)
            plsc.store_scatter(xwr_v, [e16, jnp.full((16,), GD, i32)], pe)
            for f in range(GD):
                fv = jnp.full((16,), f, i32)
                v = plsc.load_gather(xwr_v, [e16, fv])
                plsc.store_scatter(xwr_v, [e16, fv], v * pe)
        pltpu.sync_copy(xwr_v, out_hbm.at[pl.ds(off, 128)])
        return 0

    lax.fori_loop(0, EPW // 128, body, 0)


_BE = 2048
_NBE = EP // _BE


def _tc_edge_msg_body(ea_ref, xs_ref, W1_ref, b1_ref, W2t_ref, b2r_ref, S_ref,
                      msg_ref):
    h = jnp.maximum(
        jnp.dot(ea_ref[...], W1_ref[...], preferred_element_type=f32)
        + b1_ref[...], 0.0)
    t2 = jnp.dot(xs_ref[...], W2t_ref[...], preferred_element_type=f32)
    ht = pltpu.repeat(h, NFO, axis=1)
    msg = jnp.dot(t2 * ht, S_ref[...], preferred_element_type=f32)
    msg = msg + jnp.dot(xs_ref[...], b2r_ref[...], preferred_element_type=f32)
    msg_ref[...] = jnp.concatenate(
        [msg, jnp.zeros((_BE, 128 - NFO), f32)], axis=1)


def _tc_edge_msg(ea, xs, W1, b1, W2t, b2r, S):
    return pl.pallas_call(
        _tc_edge_msg_body,
        grid=(_NBE,),
        in_specs=[
            pl.BlockSpec((_BE, 4), lambda i: (i, 0)),
            pl.BlockSpec((_BE, NFI), lambda i: (i, 0)),
            pl.BlockSpec((4, MH), lambda i: (0, 0)),
            pl.BlockSpec((1, MH), lambda i: (0, 0)),
            pl.BlockSpec((NFI, NFO * MH), lambda i: (0, 0)),
            pl.BlockSpec((NFI, NFO), lambda i: (0, 0)),
            pl.BlockSpec((NFO * MH, NFO), lambda i: (0, 0)),
        ],
        out_specs=pl.BlockSpec((_BE, 128), lambda i: (i, 0)),
        out_shape=jax.ShapeDtypeStruct((EP, 128), f32),
        compiler_params=pltpu.CompilerParams(
            dimension_semantics=("arbitrary",)),
    )(ea, xs, W1, b1, W2t, b2r, S)


def _tc_node_dense_body(x_ref, agg_ref, Wr_ref, bnn_ref, Wg_ref, att_ref,
                        xw_ref, asad_ref):
    x1 = jnp.maximum(
        jnp.dot(x_ref[...], Wr_ref[...], preferred_element_type=f32)
        + agg_ref[...] + bnn_ref[...], 0.0)
    xw = jnp.dot(x1, Wg_ref[...], preferred_element_type=f32)
    asad = jnp.dot(xw, att_ref[...], preferred_element_type=f32)
    c = jnp.max(asad[:, 0]) + jnp.max(asad[:, 1])
    col = lax.broadcasted_iota(i32, (N, 4), 1)
    xw_ref[...] = xw
    asad_ref[...] = asad + jnp.where(col == 2, c, 0.0)


def _tc_node_dense(x, agg, W_root, bnn, W_gat, att2):
    return pl.pallas_call(
        _tc_node_dense_body,
        out_shape=[
            jax.ShapeDtypeStruct((N, GD), f32),
            jax.ShapeDtypeStruct((N, 4), f32),
        ],
    )(x, agg, W_root, bnn, W_gat, att2)


def _tc_gat_combine_body(tab_ref, xw_ref, asad_ref, bg_ref, x2_ref):
    tt = tab_ref[...]
    den_e = tt[:, GD:GD + 1]
    meta = asad_ref[...]
    u = meta[:, 0:1] + meta[:, 1:2]
    p_self = jnp.exp(jnp.maximum(u, 0.2 * u) - meta[:, 2:3])
    xw = xw_ref[...]
    num = tt[:, 0:GD] + p_self * xw
    den = den_e + p_self + 1e-16
    x2_ref[...] = jnp.maximum(num / den + bg_ref[...], 0.0)


def _tc_gat_combine(tab, xw, asadN, bg):
    return pl.pallas_call(
        _tc_gat_combine_body,
        out_shape=jax.ShapeDtypeStruct((N, GD), f32),
    )(tab, xw, asadN, bg)


_KB = 16000
_NKB = (N * GD) // _KB


def _tc_head_body(v_ref, W1_ref, b1_ref, W2_ref, b2_ref, out_ref, acc_ref):
    i = pl.program_id(0)

    @pl.when(i == 0)
    def _():
        acc_ref[...] = jnp.zeros_like(acc_ref)

    acc_ref[...] += jnp.dot(v_ref[0], W1_ref[...], preferred_element_type=f32)

    @pl.when(i == pl.num_programs(0) - 1)
    def _():
        v1 = jnp.maximum(acc_ref[...] + b1_ref[...], 0.0)
        out_ref[...] = jnp.maximum(
            jnp.dot(v1, W2_ref[...], preferred_element_type=f32)
            + b2_ref[...], 0.0)


def _tc_head(v3, W_fc1, b_fc1, W_fc2, b_fc2):
    return pl.pallas_call(
        _tc_head_body,
        grid=(_NKB,),
        in_specs=[
            pl.BlockSpec((1, 1, _KB), lambda i: (i, 0, 0)),
            pl.BlockSpec((_KB, HID), lambda i: (i, 0)),
            pl.BlockSpec((1, HID), lambda i: (0, 0)),
            pl.BlockSpec((HID, FIN), lambda i: (0, 0)),
            pl.BlockSpec((1, FIN), lambda i: (0, 0)),
        ],
        out_specs=pl.BlockSpec((1, FIN), lambda i: (0, 0)),
        out_shape=jax.ShapeDtypeStruct((1, FIN), f32),
        scratch_shapes=[pltpu.VMEM((1, HID), f32)],
        compiler_params=pltpu.CompilerParams(
            dimension_semantics=("arbitrary",)),
    )(v3, W_fc1, b_fc1, W_fc2, b_fc2)


def _split_idx(d):
    a = jnp.where(d < HALF, d, TD - 1)
    b = jnp.where(d >= HALF, d - HALF, TD - 1)
    return jnp.stack([a, b])


def _merge_halves(out, width):
    return jnp.concatenate(
        [out[0, :HALF, :width], out[1, :N - HALF, :width]], axis=0)


def kernel(x, edge_index, edge_attr, W1, b1, W2, b2, W_root, b_nn, W_gat,
           att_src, att_dst, b_gat, W_fc1, b_fc1, W_fc2, b_fc2):
    src = edge_index[0]
    dst = edge_index[1]

    srcP = jnp.pad(src, (0, EP - E))
    x_srcP = _sc_gather_rows(x, srcP)
    eaP = jnp.pad(edge_attr, ((0, EP - E), (0, 0)))
    W2t = W2.reshape(MH, NFI, NFO).transpose(1, 2, 0).reshape(NFI, NFO * MH)
    b2r = b2.reshape(NFI, NFO)
    S = jnp.repeat(jnp.eye(NFO, dtype=f32), MH, axis=0)
    msgP = _tc_edge_msg(eaP, x_srcP, W1, b1.reshape(1, MH), W2t, b2r, S)

    dstP = jnp.pad(dst, (0, EP - E), constant_values=NT - 1)
    aggT = _sc_scatter_rows(msgP, _split_idx(dstP))
    agg = _merge_halves(aggT, NFO)

    att2 = jnp.stack(
        [att_src, att_dst, jnp.zeros_like(att_src), jnp.zeros_like(att_src)],
        axis=1)
    xw, asadN = _tc_node_dense(x, agg, W_root, b_nn.reshape(1, NFO), W_gat,
                               att2)

    neg = jnp.float32(-1e30)
    as_col = jnp.pad(asadN[:, 0], (0, NT - N), constant_values=neg)
    xwP = jnp.concatenate(
        [jnp.pad(xw, ((0, NT - N), (0, 0))), as_col[:, None],
         jnp.zeros((NT, 128 - GD - 1), f32)], axis=1)
    adP = jnp.pad(asadN[:, 1], (0, NT - N), constant_values=neg)
    cP = asadN[:16, 2]
    siE = jnp.pad(src, (0, EP - E), constant_values=N)
    diE = jnp.pad(dst, (0, EP - E), constant_values=N)
    gatRows = _sc_gat_rows(xwP, adP, cP, siE, diE)
    gatT = _sc_scatter_rows(gatRows, _split_idx(diE))
    gat = _merge_halves(gatT, GD + 1)

    x2 = _tc_gat_combine(gat, xw, asadN, b_gat.reshape(1, GD))

    v3 = x2.reshape(_NKB, 1, _KB)
    out = _tc_head(v3, W_fc1, b_fc1.reshape(1, HID), W_fc2,
                   b_fc2.reshape(1, FIN))
    return out.reshape(FIN)


# speedup vs baseline: 6.4489x; 1.1044x over previous
import functools

import jax
import jax.numpy as jnp
from jax import lax
from jax.experimental import pallas as pl
from jax.experimental.pallas import tpu as pltpu
from jax.experimental.pallas import tpu_sc as plsc

N = 10000
E = 160000
NFI = 128
NFO = 32
MH = 32
GD = 16
HID = 256
FIN = 128

NC, NS = 2, 16
NW = NC * NS
NT = NS * 640
EP = NW * 5120
EPW = EP // NW
EPC = EP // NS
HALF = NT // 2
TD = 6144

f32 = jnp.float32
i32 = jnp.int32

_mesh = plsc.VectorSubcoreMesh(
    core_axis_name="c", subcore_axis_name="s", num_cores=NC, num_subcores=NS)


def _worker_id():
    return lax.axis_index("s") * NC + lax.axis_index("c")


def _zero_fill(ref, nrows, ncols):
    z = jnp.zeros((16,), f32)

    def row(r, _):
        for h in range(ncols // 16):
            ref[r, pl.ds(h * 16, 16)] = z
        return 0

    lax.fori_loop(0, nrows, row, 0)


@functools.partial(
    pl.kernel,
    out_type=jax.ShapeDtypeStruct((EP, NFI), f32),
    mesh=_mesh,
    scratch_types=[
        pltpu.VMEM((EPW,), i32),
        pltpu.VMEM((64, NFI), f32),
        pltpu.SemaphoreType.DMA,
    ],
    compiler_params=pltpu.CompilerParams(needs_layout_passes=False),
)
def _sc_gather_rows(x_hbm, src_hbm, out_hbm, idx_v, rows_v, sem):
    wid = _worker_id()
    base = wid * EPW
    pltpu.sync_copy(src_hbm.at[pl.ds(base, EPW)], idx_v)

    def body(j, _):
        off = j * 64
        pltpu.async_copy(
            x_hbm.at[idx_v.at[pl.ds(off, 64)]], rows_v, sem).wait()
        pltpu.sync_copy(rows_v, out_hbm.at[pl.ds(base + off, 64)])
        return 0

    lax.fori_loop(0, EPW // 64, body, 0)


@functools.partial(
    pl.kernel,
    out_type=jax.ShapeDtypeStruct((NC, TD, 128), f32),
    mesh=_mesh,
    scratch_types=[
        pltpu.VMEM((1, 128), i32),
        pltpu.VMEM((1, 128), i32),
        pltpu.VMEM((128, 128), f32),
        pltpu.VMEM((128, 128), f32),
        pltpu.VMEM((64, 128), f32),
        pltpu.VMEM_SHARED((TD, 128), f32),
        pltpu.SemaphoreType.DMA,
        pltpu.SemaphoreType.DMA,
    ],
    compiler_params=pltpu.CompilerParams(needs_layout_passes=False),
)
def _sc_scatter_rows(rows_hbm, idx2_hbm, out_hbm, idx0_v, idx1_v, rows0_v,
                     rows1_v, zb_v, table, sem0, sem1):
    cid = lax.axis_index("c")
    sid = lax.axis_index("s")
    stripe = TD // NS
    _zero_fill(zb_v, 64, 128)

    def zrow(k, _):
        pltpu.sync_copy(zb_v, table.at[pl.ds(sid * stripe + k * 64, 64)])
        return 0

    lax.fori_loop(0, stripe // 64, zrow, 0)
    plsc.subcore_barrier()
    base = sid * EPC
    NCH = EPC // 128

    def start(c, idxb, rowsb, sem):
        off = base + c * 128
        pltpu.async_copy(idx2_hbm.at[cid, pl.ds(off, 128)], idxb.at[0], sem)
        pltpu.async_copy(rows_hbm.at[pl.ds(off, 128)], rowsb, sem)

    def wait(idxb, rowsb, sem):
        pltpu.make_async_copy(
            idx2_hbm.at[cid, pl.ds(0, 128)], idxb.at[0], sem).wait()
        pltpu.make_async_copy(
            rows_hbm.at[pl.ds(0, 128)], rowsb, sem).wait()

    start(0, idx0_v, rows0_v, sem0)

    def body(j2, _):
        c0 = 2 * j2
        start(c0 + 1, idx1_v, rows1_v, sem1)
        wait(idx0_v, rows0_v, sem0)
        pltpu.sync_copy(rows0_v, table.at[idx0_v.at[0]], add=True)

        @pl.when(c0 + 2 < NCH)
        def _():
            start(c0 + 2, idx0_v, rows0_v, sem0)

        wait(idx1_v, rows1_v, sem1)
        pltpu.sync_copy(rows1_v, table.at[idx1_v.at[0]], add=True)
        return 0

    lax.fori_loop(0, NCH // 2, body, 0)
    plsc.subcore_barrier()

    def wrow(k, _):
        off = sid * stripe + k * 64
        pltpu.sync_copy(table.at[pl.ds(off, 64)], rows0_v.at[pl.ds(0, 64)])
        pltpu.sync_copy(rows0_v.at[pl.ds(0, 64)],
                        out_hbm.at[cid, pl.ds(off, 64)])
        return 0

    lax.fori_loop(0, stripe // 64, wrow, 0)


@functools.partial(
    pl.kernel,
    out_type=jax.ShapeDtypeStruct((EP, 128), f32),
    mesh=_mesh,
    scratch_types=[
        pltpu.VMEM((NT,), f32),
        pltpu.VMEM((16,), f32),
        pltpu.VMEM((1, 128), i32),
        pltpu.VMEM((1, 128), i32),
        pltpu.VMEM((128, 128), f32),
        pltpu.SemaphoreType.DMA,
    ],
    compiler_params=pltpu.CompilerParams(needs_layout_passes=False),
)
def _sc_gat_rows(xwP_hbm, adP_hbm, c_hbm, si_hbm, di_hbm, out_hbm,
                 ad_v, c_v, si_v, di_v, xwr_v, sem):
    wid = _worker_id()
    base = wid * EPW
    pltpu.sync_copy(adP_hbm, ad_v)
    pltpu.sync_copy(c_hbm, c_v)
    cvec = c_v[...]
    iota16 = lax.iota(i32, 16)

    def body(j, _):
        off = base + j * 128
        pltpu.sync_copy(si_hbm.at[pl.ds(off, 128)], si_v.at[0])
        pltpu.sync_copy(di_hbm.at[pl.ds(off, 128)], di_v.at[0])
        pltpu.async_copy(xwP_hbm.at[si_v.at[0]], xwr_v, sem).wait()
        for g in range(8):
            e16 = iota16 + g * 16
            d16 = di_v[0, pl.ds(g * 16, 16)]
            asg = plsc.load_gather(xwr_v, [e16, jnp.full((16,), GD, i32)])
            adg = plsc.load_gather(ad_v, [d16])
            u = asg + adg
            pe = jnp.exp(jnp.maximum(u, 0.2 * u) - cvec)
            plsc.store_scatter(xwr_v, [e16, jnp.full((16,), GD, i32)], pe)
            for f in range(GD):
                fv = jnp.full((16,), f, i32)
                v = plsc.load_gather(xwr_v, [e16, fv])
                plsc.store_scatter(xwr_v, [e16, fv], v * pe)
        pltpu.sync_copy(xwr_v, out_hbm.at[pl.ds(off, 128)])
        return 0

    lax.fori_loop(0, EPW // 128, body, 0)


_BE = 2048
_NBE = EP // _BE


def _tc_edge_msg_body(ea_ref, xs_ref, W1_ref, b1_ref, W2t_ref, b2r_ref, S_ref,
                      msg_ref):
    h = jnp.maximum(
        jnp.dot(ea_ref[...], W1_ref[...], preferred_element_type=f32)
        + b1_ref[...], 0.0)
    t2 = jnp.dot(xs_ref[...], W2t_ref[...], preferred_element_type=f32)
    ht = pltpu.repeat(h, NFO, axis=1)
    msg = jnp.dot(t2 * ht, S_ref[...], preferred_element_type=f32)
    msg = msg + jnp.dot(xs_ref[...], b2r_ref[...], preferred_element_type=f32)
    msg_ref[...] = jnp.concatenate(
        [msg, jnp.zeros((_BE, 128 - NFO), f32)], axis=1)


def _tc_edge_msg(ea, xs, W1, b1, W2t, b2r, S):
    return pl.pallas_call(
        _tc_edge_msg_body,
        grid=(_NBE,),
        in_specs=[
            pl.BlockSpec((_BE, 4), lambda i: (i, 0)),
            pl.BlockSpec((_BE, NFI), lambda i: (i, 0)),
            pl.BlockSpec((4, MH), lambda i: (0, 0)),
            pl.BlockSpec((1, MH), lambda i: (0, 0)),
            pl.BlockSpec((NFI, NFO * MH), lambda i: (0, 0)),
            pl.BlockSpec((NFI, NFO), lambda i: (0, 0)),
            pl.BlockSpec((NFO * MH, NFO), lambda i: (0, 0)),
        ],
        out_specs=pl.BlockSpec((_BE, 128), lambda i: (i, 0)),
        out_shape=jax.ShapeDtypeStruct((EP, 128), f32),
        compiler_params=pltpu.CompilerParams(
            dimension_semantics=("arbitrary",)),
    )(ea, xs, W1, b1, W2t, b2r, S)


def _tc_node_dense_body(x_ref, agg_ref, Wr_ref, bnn_ref, Wg_ref, att_ref,
                        xw_ref, asad_ref):
    x1 = jnp.maximum(
        jnp.dot(x_ref[...], Wr_ref[...], preferred_element_type=f32)
        + agg_ref[...] + bnn_ref[...], 0.0)
    xw = jnp.dot(x1, Wg_ref[...], preferred_element_type=f32)
    asad = jnp.dot(xw, att_ref[...], preferred_element_type=f32)
    c = jnp.max(asad[:, 0]) + jnp.max(asad[:, 1])
    col = lax.broadcasted_iota(i32, (N, 4), 1)
    xw_ref[...] = xw
    asad_ref[...] = asad + jnp.where(col == 2, c, 0.0)


def _tc_node_dense(x, agg, W_root, bnn, W_gat, att2):
    return pl.pallas_call(
        _tc_node_dense_body,
        out_shape=[
            jax.ShapeDtypeStruct((N, GD), f32),
            jax.ShapeDtypeStruct((N, 4), f32),
        ],
    )(x, agg, W_root, bnn, W_gat, att2)


def _tc_gat_combine_body(tab_ref, xw_ref, asad_ref, bg_ref, x2_ref):
    tt = tab_ref[...]
    den_e = tt[:, GD:GD + 1]
    meta = asad_ref[...]
    u = meta[:, 0:1] + meta[:, 1:2]
    p_self = jnp.exp(jnp.maximum(u, 0.2 * u) - meta[:, 2:3])
    xw = xw_ref[...]
    num = tt[:, 0:GD] + p_self * xw
    den = den_e + p_self + 1e-16
    x2_ref[...] = jnp.maximum(num / den + bg_ref[...], 0.0)


def _tc_gat_combine(tab, xw, asadN, bg):
    return pl.pallas_call(
        _tc_gat_combine_body,
        out_shape=jax.ShapeDtypeStruct((N, GD), f32),
    )(tab, xw, asadN, bg)


_KB = 16000
_NKB = (N * GD) // _KB


def _tc_head_body(v_ref, W1_ref, b1_ref, W2_ref, b2_ref, out_ref, acc_ref):
    i = pl.program_id(0)

    @pl.when(i == 0)
    def _():
        acc_ref[...] = jnp.zeros_like(acc_ref)

    acc_ref[...] += jnp.dot(v_ref[0], W1_ref[...], preferred_element_type=f32)

    @pl.when(i == pl.num_programs(0) - 1)
    def _():
        v1 = jnp.maximum(acc_ref[...] + b1_ref[...], 0.0)
        out_ref[...] = jnp.maximum(
            jnp.dot(v1, W2_ref[...], preferred_element_type=f32)
            + b2_ref[...], 0.0)


def _tc_head(v3, W_fc1, b_fc1, W_fc2, b_fc2):
    return pl.pallas_call(
        _tc_head_body,
        grid=(_NKB,),
        in_specs=[
            pl.BlockSpec((1, 1, _KB), lambda i: (i, 0, 0)),
            pl.BlockSpec((_KB, HID), lambda i: (i, 0)),
            pl.BlockSpec((1, HID), lambda i: (0, 0)),
            pl.BlockSpec((HID, FIN), lambda i: (0, 0)),
            pl.BlockSpec((1, FIN), lambda i: (0, 0)),
        ],
        out_specs=pl.BlockSpec((1, FIN), lambda i: (0, 0)),
        out_shape=jax.ShapeDtypeStruct((1, FIN), f32),
        scratch_shapes=[pltpu.VMEM((1, HID), f32)],
        compiler_params=pltpu.CompilerParams(
            dimension_semantics=("arbitrary",)),
    )(v3, W_fc1, b_fc1, W_fc2, b_fc2)


def _split_idx(d):
    a = jnp.where(d < HALF, d, TD - 1)
    b = jnp.where(d >= HALF, d - HALF, TD - 1)
    return jnp.stack([a, b])


def _merge_halves(out, width):
    return jnp.concatenate(
        [out[0, :HALF, :width], out[1, :N - HALF, :width]], axis=0)


def kernel(x, edge_index, edge_attr, W1, b1, W2, b2, W_root, b_nn, W_gat,
           att_src, att_dst, b_gat, W_fc1, b_fc1, W_fc2, b_fc2):
    src = edge_index[0]
    dst = edge_index[1]

    srcP = jnp.pad(src, (0, EP - E))
    x_srcP = _sc_gather_rows(x, srcP)
    eaP = jnp.pad(edge_attr, ((0, EP - E), (0, 0)))
    W2t = W2.reshape(MH, NFI, NFO).transpose(1, 2, 0).reshape(NFI, NFO * MH)
    b2r = b2.reshape(NFI, NFO)
    S = jnp.repeat(jnp.eye(NFO, dtype=f32), MH, axis=0)
    msgP = _tc_edge_msg(eaP, x_srcP, W1, b1.reshape(1, MH), W2t, b2r, S)

    dstP = jnp.pad(dst, (0, EP - E), constant_values=NT - 1)
    aggT = _sc_scatter_rows(msgP, _split_idx(dstP))
    agg = _merge_halves(aggT, NFO)

    att2 = jnp.stack(
        [att_src, att_dst, jnp.zeros_like(att_src), jnp.zeros_like(att_src)],
        axis=1)
    xw, asadN = _tc_node_dense(x, agg, W_root, b_nn.reshape(1, NFO), W_gat,
                               att2)

    neg = jnp.float32(-1e30)
    as_col = jnp.pad(asadN[:, 0], (0, NT - N), constant_values=neg)
    xwP = jnp.concatenate(
        [jnp.pad(xw, ((0, NT - N), (0, 0))), as_col[:, None],
         jnp.zeros((NT, 128 - GD - 1), f32)], axis=1)
    adP = jnp.pad(asadN[:, 1], (0, NT - N), constant_values=neg)
    cP = asadN[:16, 2]
    siE = jnp.pad(src, (0, EP - E), constant_values=N)
    diE = jnp.pad(dst, (0, EP - E), constant_values=N)
    gatRows = _sc_gat_rows(xwP, adP, cP, siE, diE)
    gatT = _sc_scatter_rows(gatRows, _split_idx(diE))
    gat = _merge_halves(gatT, GD + 1)

    x2 = _tc_gat_combine(gat, xw, asadN, b_gat.reshape(1, GD))

    v3 = x2.reshape(_NKB, 1, _KB)
    out = _tc_head(v3, W_fc1, b_fc1.reshape(1, HID), W_fc2,
                   b_fc2.reshape(1, FIN))
    return out.reshape(FIN)


# speedup vs baseline: 7.3793x; 1.1443x over previous
import functools

import jax
import jax.numpy as jnp
from jax import lax
from jax.experimental import pallas as pl
from jax.experimental.pallas import tpu as pltpu
from jax.experimental.pallas import tpu_sc as plsc

N = 10000
E = 160000
NFI = 128
NFO = 32
MH = 32
GD = 16
HID = 256
FIN = 128

NC, NS = 2, 16
NW = NC * NS
NT = NS * 640
EP = NW * 5120
EPW = EP // NW
EPC = EP // NS
HALF = NT // 2
TD = 6144

f32 = jnp.float32
i32 = jnp.int32

_mesh = plsc.VectorSubcoreMesh(
    core_axis_name="c", subcore_axis_name="s", num_cores=NC, num_subcores=NS)


def _worker_id():
    return lax.axis_index("s") * NC + lax.axis_index("c")


def _zero_fill(ref, nrows, ncols):
    z = jnp.zeros((16,), f32)

    def row(r, _):
        for h in range(ncols // 16):
            ref[r, pl.ds(h * 16, 16)] = z
        return 0

    lax.fori_loop(0, nrows, row, 0)


@functools.partial(
    pl.kernel,
    out_type=jax.ShapeDtypeStruct((EP, NFI), f32),
    mesh=_mesh,
    scratch_types=[
        pltpu.VMEM((EPW,), i32),
        pltpu.VMEM((64, NFI), f32),
        pltpu.VMEM((64, NFI), f32),
        pltpu.SemaphoreType.DMA,
        pltpu.SemaphoreType.DMA,
    ],
    compiler_params=pltpu.CompilerParams(needs_layout_passes=False),
)
def _sc_gather_rows(x_hbm, src_hbm, out_hbm, idx_v, rows0_v, rows1_v, sem0,
                    sem1):
    wid = _worker_id()
    base = wid * EPW
    pltpu.sync_copy(src_hbm.at[pl.ds(base, EPW)], idx_v)
    NCH = EPW // 64

    def start(j, rowsb, sem):
        pltpu.async_copy(x_hbm.at[idx_v.at[pl.ds(j * 64, 64)]], rowsb, sem)

    def wait(rowsb, sem):
        pltpu.make_async_copy(x_hbm.at[pl.ds(0, 64)], rowsb, sem).wait()

    start(0, rows0_v, sem0)

    def body(j2, _):
        j0 = 2 * j2
        start(j0 + 1, rows1_v, sem1)
        wait(rows0_v, sem0)
        pltpu.sync_copy(rows0_v, out_hbm.at[pl.ds(base + j0 * 64, 64)])

        @pl.when(j0 + 2 < NCH)
        def _():
            start(j0 + 2, rows0_v, sem0)

        wait(rows1_v, sem1)
        pltpu.sync_copy(rows1_v, out_hbm.at[pl.ds(base + (j0 + 1) * 64, 64)])
        return 0

    lax.fori_loop(0, NCH // 2, body, 0)


@functools.partial(
    pl.kernel,
    out_type=jax.ShapeDtypeStruct((NC, TD, 128), f32),
    mesh=_mesh,
    scratch_types=[
        pltpu.VMEM((1, 128), i32),
        pltpu.VMEM((1, 128), i32),
        pltpu.VMEM((128, 128), f32),
        pltpu.VMEM((128, 128), f32),
        pltpu.VMEM((32, 128), f32),
        pltpu.VMEM_SHARED((TD, 128), f32),
        pltpu.SemaphoreType.DMA,
        pltpu.SemaphoreType.DMA,
    ],
    compiler_params=pltpu.CompilerParams(needs_layout_passes=False),
)
def _sc_scatter_rows(rows_hbm, idx2_hbm, out_hbm, idx0_v, idx1_v, rows0_v,
                     rows1_v, zb_v, table, sem0, sem1):
    cid = lax.axis_index("c")
    sid = lax.axis_index("s")
    stripe = TD // NS
    _zero_fill(zb_v, 32, 128)

    def zrow(k, _):
        pltpu.sync_copy(zb_v, table.at[pl.ds(sid * stripe + k * 32, 32)])
        return 0

    lax.fori_loop(0, stripe // 32, zrow, 0)
    plsc.subcore_barrier()
    base = sid * EPC
    NCH = EPC // 128

    def start(c, idxb, rowsb, sem):
        off = base + c * 128
        pltpu.async_copy(idx2_hbm.at[cid, pl.ds(off, 128)], idxb.at[0], sem)
        pltpu.async_copy(rows_hbm.at[pl.ds(off, 128)], rowsb, sem)

    def wait(idxb, rowsb, sem):
        pltpu.make_async_copy(
            idx2_hbm.at[cid, pl.ds(0, 128)], idxb.at[0], sem).wait()
        pltpu.make_async_copy(
            rows_hbm.at[pl.ds(0, 128)], rowsb, sem).wait()

    start(0, idx0_v, rows0_v, sem0)

    def body(j2, _):
        c0 = 2 * j2
        start(c0 + 1, idx1_v, rows1_v, sem1)
        wait(idx0_v, rows0_v, sem0)
        pltpu.sync_copy(rows0_v, table.at[idx0_v.at[0]], add=True)

        @pl.when(c0 + 2 < NCH)
        def _():
            start(c0 + 2, idx0_v, rows0_v, sem0)

        wait(idx1_v, rows1_v, sem1)
        pltpu.sync_copy(rows1_v, table.at[idx1_v.at[0]], add=True)
        return 0

    lax.fori_loop(0, NCH // 2, body, 0)
    plsc.subcore_barrier()

    def wrow(k, _):
        off = sid * stripe + k * 64
        pltpu.sync_copy(table.at[pl.ds(off, 64)], rows0_v.at[pl.ds(0, 64)])
        pltpu.sync_copy(rows0_v.at[pl.ds(0, 64)],
                        out_hbm.at[cid, pl.ds(off, 64)])
        return 0

    lax.fori_loop(0, stripe // 64, wrow, 0)


@functools.partial(
    pl.kernel,
    out_type=jax.ShapeDtypeStruct((EP, 128), f32),
    mesh=_mesh,
    scratch_types=[
        pltpu.VMEM((NT,), f32),
        pltpu.VMEM((16,), f32),
        pltpu.VMEM((EPW,), i32),
        pltpu.VMEM((EPW,), i32),
        pltpu.VMEM((128, 128), f32),
        pltpu.VMEM((128, 128), f32),
        pltpu.SemaphoreType.DMA,
        pltpu.SemaphoreType.DMA,
    ],
    compiler_params=pltpu.CompilerParams(needs_layout_passes=False),
)
def _sc_gat_rows(xwP_hbm, adP_hbm, c_hbm, si_hbm, di_hbm, out_hbm,
                 ad_v, c_v, si_v, di_v, xwr0_v, xwr1_v, sem0, sem1):
    wid = _worker_id()
    base = wid * EPW
    pltpu.sync_copy(adP_hbm, ad_v)
    pltpu.sync_copy(c_hbm, c_v)
    pltpu.sync_copy(si_hbm.at[pl.ds(base, EPW)], si_v)
    pltpu.sync_copy(di_hbm.at[pl.ds(base, EPW)], di_v)
    cvec = c_v[...]
    iota16 = lax.iota(i32, 16)
    NCH = EPW // 128

    def start(j, xwrb, sem):
        pltpu.async_copy(
            xwP_hbm.at[si_v.at[pl.ds(j * 128, 128)]], xwrb, sem)

    def wait(xwrb, sem):
        pltpu.make_async_copy(xwP_hbm.at[pl.ds(0, 128)], xwrb, sem).wait()

    def process(j, xwrb):
        off = j * 128
        for g in range(8):
            e16 = iota16 + g * 16
            d16 = di_v[pl.ds(off + g * 16, 16)]
            asg = plsc.load_gather(xwrb, [e16, jnp.full((16,), GD, i32)])
            adg = plsc.load_gather(ad_v, [d16])
            u = asg + adg
            pe = jnp.exp(jnp.maximum(u, 0.2 * u) - cvec)
            plsc.store_scatter(xwrb, [e16, jnp.full((16,), GD, i32)], pe)
            for f in range(GD):
                fv = jnp.full((16,), f, i32)
                v = plsc.load_gather(xwrb, [e16, fv])
                plsc.store_scatter(xwrb, [e16, fv], v * pe)
        pltpu.sync_copy(xwrb, out_hbm.at[pl.ds(base + off, 128)])

    start(0, xwr0_v, sem0)

    def body(j2, _):
        j0 = 2 * j2
        start(j0 + 1, xwr1_v, sem1)
        wait(xwr0_v, sem0)
        process(j0, xwr0_v)

        @pl.when(j0 + 2 < NCH)
        def _():
            start(j0 + 2, xwr0_v, sem0)

        wait(xwr1_v, sem1)
        process(j0 + 1, xwr1_v)
        return 0

    lax.fori_loop(0, NCH // 2, body, 0)


_BE = 2048
_NBE = EP // _BE


def _tc_edge_msg_body(ea_ref, xs_ref, W1_ref, b1_ref, W2t_ref, b2r_ref, S_ref,
                      msg_ref):
    h = jnp.maximum(
        jnp.dot(ea_ref[...], W1_ref[...], preferred_element_type=f32)
        + b1_ref[...], 0.0)
    t2 = jnp.dot(xs_ref[...], W2t_ref[...], preferred_element_type=f32)
    ht = pltpu.repeat(h, NFO, axis=1)
    msg = jnp.dot(t2 * ht, S_ref[...], preferred_element_type=f32)
    msg = msg + jnp.dot(xs_ref[...], b2r_ref[...], preferred_element_type=f32)
    msg_ref[...] = jnp.concatenate(
        [msg, jnp.zeros((_BE, 128 - NFO), f32)], axis=1)


def _tc_edge_msg(ea, xs, W1, b1, W2t, b2r, S):
    return pl.pallas_call(
        _tc_edge_msg_body,
        grid=(_NBE,),
        in_specs=[
            pl.BlockSpec((_BE, 4), lambda i: (i, 0)),
            pl.BlockSpec((_BE, NFI), lambda i: (i, 0)),
            pl.BlockSpec((4, MH), lambda i: (0, 0)),
            pl.BlockSpec((1, MH), lambda i: (0, 0)),
            pl.BlockSpec((NFI, NFO * MH), lambda i: (0, 0)),
            pl.BlockSpec((NFI, NFO), lambda i: (0, 0)),
            pl.BlockSpec((NFO * MH, NFO), lambda i: (0, 0)),
        ],
        out_specs=pl.BlockSpec((_BE, 128), lambda i: (i, 0)),
        out_shape=jax.ShapeDtypeStruct((EP, 128), f32),
        compiler_params=pltpu.CompilerParams(
            dimension_semantics=("arbitrary",)),
    )(ea, xs, W1, b1, W2t, b2r, S)


def _tc_node_dense_body(x_ref, agg_ref, Wr_ref, bnn_ref, Wg_ref, att_ref,
                        xw_ref, asad_ref):
    x1 = jnp.maximum(
        jnp.dot(x_ref[...], Wr_ref[...], preferred_element_type=f32)
        + agg_ref[...] + bnn_ref[...], 0.0)
    xw = jnp.dot(x1, Wg_ref[...], preferred_element_type=f32)
    asad = jnp.dot(xw, att_ref[...], preferred_element_type=f32)
    c = jnp.max(asad[:, 0]) + jnp.max(asad[:, 1])
    col = lax.broadcasted_iota(i32, (N, 4), 1)
    xw_ref[...] = xw
    asad_ref[...] = asad + jnp.where(col == 2, c, 0.0)


def _tc_node_dense(x, agg, W_root, bnn, W_gat, att2):
    return pl.pallas_call(
        _tc_node_dense_body,
        out_shape=[
            jax.ShapeDtypeStruct((N, GD), f32),
            jax.ShapeDtypeStruct((N, 4), f32),
        ],
    )(x, agg, W_root, bnn, W_gat, att2)


def _tc_gat_combine_body(tab_ref, xw_ref, asad_ref, bg_ref, x2_ref):
    tt = tab_ref[...]
    den_e = tt[:, GD:GD + 1]
    meta = asad_ref[...]
    u = meta[:, 0:1] + meta[:, 1:2]
    p_self = jnp.exp(jnp.maximum(u, 0.2 * u) - meta[:, 2:3])
    xw = xw_ref[...]
    num = tt[:, 0:GD] + p_self * xw
    den = den_e + p_self + 1e-16
    x2_ref[...] = jnp.maximum(num / den + bg_ref[...], 0.0)


def _tc_gat_combine(tab, xw, asadN, bg):
    return pl.pallas_call(
        _tc_gat_combine_body,
        out_shape=jax.ShapeDtypeStruct((N, GD), f32),
    )(tab, xw, asadN, bg)


_KB = 16000
_NKB = (N * GD) // _KB


def _tc_head_body(v_ref, W1_ref, b1_ref, W2_ref, b2_ref, out_ref, acc_ref):
    i = pl.program_id(0)

    @pl.when(i == 0)
    def _():
        acc_ref[...] = jnp.zeros_like(acc_ref)

    acc_ref[...] += jnp.dot(v_ref[0], W1_ref[...], preferred_element_type=f32)

    @pl.when(i == pl.num_programs(0) - 1)
    def _():
        v1 = jnp.maximum(acc_ref[...] + b1_ref[...], 0.0)
        out_ref[...] = jnp.maximum(
            jnp.dot(v1, W2_ref[...], preferred_element_type=f32)
            + b2_ref[...], 0.0)


def _tc_head(v3, W_fc1, b_fc1, W_fc2, b_fc2):
    return pl.pallas_call(
        _tc_head_body,
        grid=(_NKB,),
        in_specs=[
            pl.BlockSpec((1, 1, _KB), lambda i: (i, 0, 0)),
            pl.BlockSpec((_KB, HID), lambda i: (i, 0)),
            pl.BlockSpec((1, HID), lambda i: (0, 0)),
            pl.BlockSpec((HID, FIN), lambda i: (0, 0)),
            pl.BlockSpec((1, FIN), lambda i: (0, 0)),
        ],
        out_specs=pl.BlockSpec((1, FIN), lambda i: (0, 0)),
        out_shape=jax.ShapeDtypeStruct((1, FIN), f32),
        scratch_shapes=[pltpu.VMEM((1, HID), f32)],
        compiler_params=pltpu.CompilerParams(
            dimension_semantics=("arbitrary",)),
    )(v3, W_fc1, b_fc1, W_fc2, b_fc2)


def _split_idx(d):
    a = jnp.where(d < HALF, d, TD - 1)
    b = jnp.where(d >= HALF, d - HALF, TD - 1)
    return jnp.stack([a, b])


def _merge_halves(out, width):
    return jnp.concatenate(
        [out[0, :HALF, :width], out[1, :N - HALF, :width]], axis=0)


def kernel(x, edge_index, edge_attr, W1, b1, W2, b2, W_root, b_nn, W_gat,
           att_src, att_dst, b_gat, W_fc1, b_fc1, W_fc2, b_fc2):
    src = edge_index[0]
    dst = edge_index[1]

    srcP = jnp.pad(src, (0, EP - E))
    x_srcP = _sc_gather_rows(x, srcP)
    eaP = jnp.pad(edge_attr, ((0, EP - E), (0, 0)))
    W2t = W2.reshape(MH, NFI, NFO).transpose(1, 2, 0).reshape(NFI, NFO * MH)
    b2r = b2.reshape(NFI, NFO)
    S = jnp.repeat(jnp.eye(NFO, dtype=f32), MH, axis=0)
    msgP = _tc_edge_msg(eaP, x_srcP, W1, b1.reshape(1, MH), W2t, b2r, S)

    dstP = jnp.pad(dst, (0, EP - E), constant_values=NT - 1)
    aggT = _sc_scatter_rows(msgP, _split_idx(dstP))
    agg = _merge_halves(aggT, NFO)

    att2 = jnp.stack(
        [att_src, att_dst, jnp.zeros_like(att_src), jnp.zeros_like(att_src)],
        axis=1)
    xw, asadN = _tc_node_dense(x, agg, W_root, b_nn.reshape(1, NFO), W_gat,
                               att2)

    neg = jnp.float32(-1e30)
    as_col = jnp.pad(asadN[:, 0], (0, NT - N), constant_values=neg)
    xwP = jnp.concatenate(
        [jnp.pad(xw, ((0, NT - N), (0, 0))), as_col[:, None],
         jnp.zeros((NT, 128 - GD - 1), f32)], axis=1)
    adP = jnp.pad(asadN[:, 1], (0, NT - N), constant_values=neg)
    cP = asadN[:16, 2]
    siE = jnp.pad(src, (0, EP - E), constant_values=N)
    diE = jnp.pad(dst, (0, EP - E), constant_values=N)
    gatRows = _sc_gat_rows(xwP, adP, cP, siE, diE)
    gatT = _sc_scatter_rows(gatRows, _split_idx(diE))
    gat = _merge_halves(gatT, GD + 1)

    x2 = _tc_gat_combine(gat, xw, asadN, b_gat.reshape(1, GD))

    v3 = x2.reshape(_NKB, 1, _KB)
    out = _tc_head(v3, W_fc1, b_fc1.reshape(1, HID), W_fc2,
                   b_fc2.reshape(1, FIN))
    return out.reshape(FIN)
